# baseline (speedup 1.0000x reference)
import dataclasses
import functools

import jax
import jax.numpy as jnp
from jax import lax
from jax.experimental import pallas as pl
from jax.experimental.pallas import tpu as pltpu
from jax.experimental.pallas import tpu_sc as plsc

_NC = 2
_NS = 16
_NW = _NC * _NS
_L = 16

_CS = 128


def _c16(v):
    return jnp.full((_L,), v, dtype=jnp.int32)


def _sc_gather(memory, lu16, t, n_id, src, pos_dst, neg_dst):
    V, D = memory.shape
    (N,) = n_id.shape
    (B,) = t.shape
    assert B % (_NW * _CS) == 0
    pw = B // _NW
    nch = pw // _CS
    mesh = plsc.VectorSubcoreMesh(core_axis_name="c", subcore_axis_name="s")
    cp = pltpu.CompilerParams()
    fields = pltpu.CompilerParams.__dataclass_fields__
    if "needs_layout_passes" in fields:
        cp = dataclasses.replace(cp, needs_layout_passes=False)
    if "use_tc_tiling_on_sc" in fields:
        cp = dataclasses.replace(cp, use_tc_tiling_on_sc=False)

    rows_t = jax.ShapeDtypeStruct((B, D), jnp.float32)
    rel_t = jax.ShapeDtypeStruct((B,), jnp.float32)

    @functools.partial(
        pl.kernel,
        out_type=[rows_t, rows_t, rows_t, rel_t, rel_t, rel_t],
        mesh=mesh,
        compiler_params=cp,
        scratch_types=[
            pltpu.VMEM((N,), jnp.int32),
            pltpu.VMEM((pw,), jnp.float32),
            pltpu.VMEM((pw,), jnp.int32),
            pltpu.VMEM((nch, _CS), jnp.int32),
            pltpu.VMEM((nch, _CS), jnp.int32),
            pltpu.VMEM((nch, _CS), jnp.int32),
            pltpu.VMEM((_CS, D), jnp.float32),
            pltpu.VMEM((_CS, 16), jnp.float32),
            pltpu.VMEM((_CS,), jnp.float32),
            pltpu.SemaphoreType.DMA,
            pltpu.SemaphoreType.DMA,
        ],
    )
    def sc_kernel(mem_h, lu16_h, t_h, nid_h, src_h, pos_h, neg_h,
                  msrc_o, mpos_o, mneg_o, rs_o, rp_o, rn_o,
                  nid_v, t_v, sidx_v, g_v, r_v, l_v, rows_v, lurow_v, rel_v,
                  sem1, sem2):
        wid = lax.axis_index("s") * _NC + lax.axis_index("c")
        base = wid * pw
        pltpu.sync_copy(nid_h, nid_v)
        pltpu.sync_copy(t_h.at[pl.ds(base, pw)], t_v)
        for idx_h, m_o, rel_o in ((src_h, msrc_o, rs_o),
                                  (pos_h, mpos_o, rp_o),
                                  (neg_h, mneg_o, rn_o)):
            pltpu.sync_copy(idx_h.at[pl.ds(base, pw)], sidx_v)
            for i in range(pw // _L):
                iv = sidx_v[pl.ds(i * _L, _L)]
                g = plsc.load_gather(nid_v, [iv])
                j, off = divmod(i * _L, _CS)
                g_v[j, pl.ds(off, _L)] = g
                r_v[j, pl.ds(off, _L)] = lax.shift_right_logical(g, _c16(4))
                l_v[j, pl.ds(off, _L)] = lax.bitwise_and(g, _c16(15))
            for j in range(nch):
                cm = pltpu.async_copy(mem_h.at[g_v.at[j]], rows_v, sem1)
                cl = pltpu.async_copy(lu16_h.at[r_v.at[j]], lurow_v, sem2)
                cm.wait()
                cl.wait()
                for k in range(_CS // _L):
                    rows16 = lax.iota(jnp.int32, _L) + _c16(k * _L)
                    lanes = l_v[j, pl.ds(k * _L, _L)]
                    luv = plsc.load_gather(lurow_v, [rows16, lanes])
                    tv = t_v[pl.ds(j * _CS + k * _L, _L)]
                    rel_v[pl.ds(k * _L, _L)] = jnp.abs(luv - tv)
                pltpu.sync_copy(rows_v, m_o.at[pl.ds(base + j * _CS, _CS)])
                pltpu.sync_copy(rel_v, rel_o.at[pl.ds(base + j * _CS, _CS)])

    return sc_kernel(memory, lu16, t, n_id, src, pos_dst, neg_dst)


def _tc_body(ms, mp, mn, rs, rp, rn, wps, bps, wpd, bpd,
             wlsT, bls, wldT, bld, wf, bf, pos_o, neg_o):
    dn = (((1,), (0,)), ((), ()))
    z_s = ms[...] * (1.0 + rs[...] * wps[...] + bps[...])
    z_p = mp[...] * (1.0 + rp[...] * wpd[...] + bpd[...])
    z_n = mn[...] * (1.0 + rn[...] * wpd[...] + bpd[...])
    h_s = lax.dot_general(z_s, wlsT[...], dn,
                          preferred_element_type=jnp.float32) + bls[...]
    h_p = lax.dot_general(z_p, wldT[...], dn,
                          preferred_element_type=jnp.float32) + bld[...]
    h_n = lax.dot_general(z_n, wldT[...], dn,
                          preferred_element_type=jnp.float32) + bld[...]
    q_p = jnp.maximum(h_s + h_p, 0.0)
    q_n = jnp.maximum(h_s + h_n, 0.0)
    pos_o[...] = jnp.sum(q_p * wf[...], axis=1, keepdims=True) + bf[...]
    neg_o[...] = jnp.sum(q_n * wf[...], axis=1, keepdims=True) + bf[...]


def _tc_compute(ms, mp, mn, rs, rp, rn, wps, bps, wpd, bpd,
                wlsT, bls, wldT, bld, wf, bf):
    B, D = ms.shape
    blk = 2048
    grid = B // blk
    row_spec = pl.BlockSpec((blk, D), lambda i: (i, 0))
    col_spec = pl.BlockSpec((blk, 1), lambda i: (i, 0))

    def w_spec(a):
        return pl.BlockSpec(a.shape, lambda i: (0, 0))

    out_sd = jax.ShapeDtypeStruct((B, 1), jnp.float32)
    return pl.pallas_call(
        _tc_body,
        grid=(grid,),
        in_specs=[row_spec, row_spec, row_spec, col_spec, col_spec, col_spec,
                  w_spec(wps), w_spec(bps), w_spec(wpd), w_spec(bpd),
                  w_spec(wlsT), w_spec(bls), w_spec(wldT), w_spec(bld),
                  w_spec(wf), w_spec(bf)],
        out_specs=[col_spec, col_spec],
        out_shape=[out_sd, out_sd],
    )(ms, mp, mn, rs, rp, rn, wps, bps, wpd, bpd,
      wlsT, bls, wldT, bld, wf, bf)


def kernel(memory, last_update, t, n_id, src, pos_dst, neg_dst,
           W_proj_src, b_proj_src, W_proj_dst, b_proj_dst,
           W_lin_src, b_lin_src, W_lin_dst, b_lin_dst, W_final, b_final):
    V, D = memory.shape
    B = t.shape[0]
    lu16 = last_update.reshape(V // 16, 16)
    n_id = n_id.astype(jnp.int32)
    src = src.astype(jnp.int32)
    pos_dst = pos_dst.astype(jnp.int32)
    neg_dst = neg_dst.astype(jnp.int32)

    m_src, m_pos, m_neg, rel_s, rel_p, rel_n = _sc_gather(
        memory, lu16, t, n_id, src, pos_dst, neg_dst)

    pos_out, neg_out = _tc_compute(
        m_src, m_pos, m_neg,
        rel_s.reshape(B, 1), rel_p.reshape(B, 1), rel_n.reshape(B, 1),
        W_proj_src.reshape(1, D), b_proj_src.reshape(1, D),
        W_proj_dst.reshape(1, D), b_proj_dst.reshape(1, D),
        W_lin_src.T, b_lin_src.reshape(1, D),
        W_lin_dst.T, b_lin_dst.reshape(1, D),
        W_final, b_final.reshape(1, 1))

    return (pos_out, neg_out, m_src, m_pos)
